# baseline (speedup 1.0000x reference)
import functools

import jax
import jax.numpy as jnp
from jax import lax
from jax.experimental import pallas as pl
from jax.experimental.pallas import tpu as pltpu
from jax.experimental.pallas import tpu_sc as plsc

NEG = 0.2
LANES = 16
CHUNK = 64
NSUB = 16
NCORE = 2
NBUF = 3


def _prep_body(x_ref, w_ref, asr_ref, adr_ref, hp_ref, adw_ref):
    h = jnp.dot(x_ref[...], w_ref[...], preferred_element_type=jnp.float32,
                precision=lax.Precision.HIGHEST)
    asrc = jnp.sum(h * asr_ref[...], axis=1)
    adst = jnp.sum(h * adr_ref[...], axis=1)
    n = h.shape[0]
    hp_ref[...] = jnp.concatenate(
        [h, jnp.broadcast_to(asrc[:, None], (n, LANES))], axis=1)
    adw_ref[...] = jnp.broadcast_to(adst[:, None], (n, LANES))


def _finish_body(acc_ref, hp_ref, adw_ref, b_ref, o_ref):
    nf = acc_ref.shape[2] - LANES
    h = hp_ref[:, :nf]
    e = hp_ref[:, nf] + adw_ref[:, 0]
    e = jnp.where(e >= 0.0, e, NEG * e)
    ps = jnp.exp(e)
    hs = acc_ref[0, :, :nf] + acc_ref[1, :, :nf] + ps[:, None] * h
    dn = acc_ref[0, :, nf] + acc_ref[1, :, nf] + ps
    y = hs / dn[:, None] + b_ref[...]
    o_ref[...] = jnp.where(y >= 0.0, y, NEG * y)


def _make_edge_kernel(n_nodes, featp, t_chunks, e_tot):
    mesh = plsc.VectorSubcoreMesh(core_axis_name="c", subcore_axis_name="s")

    @functools.partial(
        pl.kernel,
        mesh=mesh,
        compiler_params=pltpu.CompilerParams(
            needs_layout_passes=False, use_tc_tiling_on_sc=False),
        out_type=jax.ShapeDtypeStruct((NCORE, n_nodes, featp), jnp.float32),
        scratch_types=[
            [pltpu.VMEM((CHUNK,), jnp.int32) for _ in range(NBUF)],
            [pltpu.VMEM((CHUNK,), jnp.int32) for _ in range(NBUF)],
            [pltpu.VMEM((CHUNK,), jnp.int32) for _ in range(NBUF)],
            [pltpu.VMEM((CHUNK, featp), jnp.float32) for _ in range(NBUF)],
            [pltpu.VMEM((CHUNK, LANES), jnp.float32) for _ in range(NBUF)],
            pltpu.VMEM_SHARED((n_nodes, featp), jnp.float32),
            [pltpu.SemaphoreType.DMA for _ in range(NBUF)],
            [pltpu.SemaphoreType.DMA for _ in range(NBUF)],
            [pltpu.SemaphoreType.DMA for _ in range(NBUF)],
        ],
    )
    def edge_kernel(hp_hbm, adw_hbm, idx_hbm, out_hbm,
                    s_v, d_v, ds_v, r_v, b_v, acc, sem_g, sem_i, sem_s):
        c = lax.axis_index("c")
        s = lax.axis_index("s")

        @plsc.parallel_loop(0, CHUNK, 1, unroll=4)
        def _(i):
            for f in range(featp // LANES):
                r_v[0][i, pl.ds(f * LANES, LANES)] = jnp.zeros(
                    (LANES,), jnp.float32)

        npt = n_nodes // NSUB
        nfull = npt // CHUNK
        nrem = npt - nfull * CHUNK
        row0 = s * npt
        for q in range(nfull):
            pltpu.sync_copy(r_v[0], acc.at[pl.ds(row0 + q * CHUNK, CHUNK)])
        if nrem:
            pltpu.sync_copy(r_v[0].at[pl.ds(0, nrem)],
                            acc.at[pl.ds(row0 + nfull * CHUNK, nrem)])
        plsc.subcore_barrier()

        tile_base = (c * NSUB + s) * (t_chunks * CHUNK)

        def issue_gathers(b):
            pltpu.async_copy(hp_hbm.at[s_v[b]], r_v[b], sem_g[b])
            pltpu.async_copy(adw_hbm.at[d_v[b]], b_v[b], sem_g[b])

        def wait_gathers(b):
            pltpu.make_async_copy(hp_hbm.at[s_v[b]], r_v[b], sem_g[b]).wait()
            pltpu.make_async_copy(adw_hbm.at[d_v[b]], b_v[b], sem_g[b]).wait()

        def issue_idx(j, b):
            pltpu.async_copy(idx_hbm.at[0, c, s, j], s_v[b], sem_i[b])
            pltpu.async_copy(idx_hbm.at[1, c, s, j], d_v[b], sem_i[b])

        def wait_idx(j, b):
            pltpu.make_async_copy(idx_hbm.at[0, c, s, j], s_v[b], sem_i[b]).wait()
            pltpu.make_async_copy(idx_hbm.at[1, c, s, j], d_v[b], sem_i[b]).wait()

        def issue_scatter(b):
            pltpu.async_copy(r_v[b], acc.at[ds_v[b]], sem_s[b], add=True)

        def wait_scatter(b):
            pltpu.make_async_copy(r_v[b], acc.at[ds_v[b]], sem_s[b]).wait()

        def compute(j, b):
            gbase = tile_base + j * CHUNK

            @plsc.parallel_loop(0, CHUNK, 1, unroll=4)
            def _(i):
                e = r_v[b][i, pl.ds(featp - LANES, LANES)] + b_v[b][i]
                e = jnp.where(e >= 0.0, e, NEG * e)
                p = jnp.exp(e)
                gv = jnp.full((LANES,), gbase + i, jnp.int32)
                p = jnp.where(gv < e_tot, p, 0.0)
                for f in range((featp - LANES) // LANES):
                    r_v[b][i, pl.ds(f * LANES, LANES)] = (
                        r_v[b][i, pl.ds(f * LANES, LANES)] * p)
                r_v[b][i, pl.ds(featp - LANES, LANES)] = jnp.where(
                    lax.iota(jnp.int32, LANES) == 0, p, 0.0)

        pltpu.sync_copy(idx_hbm.at[0, c, s, 0], s_v[0])
        pltpu.sync_copy(idx_hbm.at[1, c, s, 0], d_v[0])
        issue_gathers(0)
        issue_idx(1, 1)

        nt = t_chunks // NBUF

        def trio_body(jj, carry):
            for b in range(NBUF):
                pn = (b + 1) % NBUF
                pp = (b + 2) % NBUF
                j = jj * NBUF + b
                if b < NBUF - 1:
                    wait_idx(j + 1, pn)
                else:
                    @pl.when(jj < nt - 1)
                    def _():
                        wait_idx(j + 1, pn)
                if b == 0:
                    @pl.when(jj > 0)
                    def _():
                        wait_scatter(pn)
                elif b == 1:
                    @pl.when(jj > 0)
                    def _():
                        wait_scatter(pn)
                else:
                    wait_scatter(pn)
                if b < NBUF - 1:
                    issue_gathers(pn)
                else:
                    @pl.when(jj < nt - 1)
                    def _():
                        issue_gathers(pn)
                if b == 0:
                    issue_idx(j + 2, pp)
                else:
                    @pl.when(jj < nt - 1)
                    def _():
                        issue_idx(j + 2, pp)
                wait_gathers(b)
                for u in range(CHUNK // LANES):
                    ds_v[b][pl.ds(u * LANES, LANES)] = (
                        d_v[b][pl.ds(u * LANES, LANES)])
                compute(j, b)
                issue_scatter(b)
            return carry

        lax.fori_loop(0, nt, trio_body, 0)
        wait_scatter((t_chunks - 2) % NBUF)
        wait_scatter((t_chunks - 1) % NBUF)
        plsc.subcore_barrier()

        pltpu.sync_copy(acc.at[pl.ds(row0, npt)],
                        out_hbm.at[c, pl.ds(row0, npt)])

    return edge_kernel


def kernel(x, edge_index, W, att_src, att_dst, bias):
    n_nodes, in_f = x.shape
    out_f = W.shape[1]
    featp = out_f + LANES
    n_edges = edge_index.shape[1]
    n_workers = NCORE * NSUB
    t_chunks = -(-n_edges // (n_workers * CHUNK))
    t_chunks += (-t_chunks) % NBUF
    e_pad = n_workers * t_chunks * CHUNK

    idx = jnp.pad(edge_index.astype(jnp.int32), ((0, 0), (0, e_pad - n_edges))
                  ).reshape(2, NCORE, NSUB, t_chunks, CHUNK)

    hp, adw = pl.pallas_call(
        _prep_body,
        out_shape=[
            jax.ShapeDtypeStruct((n_nodes, featp), jnp.float32),
            jax.ShapeDtypeStruct((n_nodes, LANES), jnp.float32),
        ],
    )(x, W, att_src.reshape(1, out_f), att_dst.reshape(1, out_f))

    acc = _make_edge_kernel(n_nodes, featp, t_chunks, n_edges)(hp, adw, idx)

    grid = 10
    rblk = n_nodes // grid
    out = pl.pallas_call(
        _finish_body,
        grid=(grid,),
        in_specs=[
            pl.BlockSpec((NCORE, rblk, featp), lambda i: (0, i, 0)),
            pl.BlockSpec((rblk, featp), lambda i: (i, 0)),
            pl.BlockSpec((rblk, LANES), lambda i: (i, 0)),
            pl.BlockSpec((1, out_f), lambda i: (0, 0)),
        ],
        out_specs=pl.BlockSpec((rblk, out_f), lambda i: (i, 0)),
        out_shape=jax.ShapeDtypeStruct((n_nodes, out_f), jnp.float32),
    )(acc, hp, adw, bias.reshape(1, out_f))
    return out


# speedup vs baseline: 1.4180x; 1.4180x over previous
import functools

import jax
import jax.numpy as jnp
from jax import lax
from jax.experimental import pallas as pl
from jax.experimental.pallas import tpu as pltpu
from jax.experimental.pallas import tpu_sc as plsc

NEG = 0.2
LANES = 16
CHUNK = 64
NSUB = 16
NCORE = 2
NBUF = 3


def _prep_body(x_ref, w_ref, asr_ref, adr_ref, hp_ref, adw_ref):
    h = jnp.dot(x_ref[...], w_ref[...], preferred_element_type=jnp.float32,
                precision=lax.Precision.HIGHEST)
    asrc = jnp.sum(h * asr_ref[...], axis=1)
    adst = jnp.sum(h * adr_ref[...], axis=1)
    n = h.shape[0]
    hp_ref[...] = jnp.concatenate(
        [h, jnp.broadcast_to(asrc[:, None], (n, LANES))], axis=1)
    adw_ref[...] = jnp.broadcast_to(adst[:, None], (n, LANES))


def _finish_body(acc_ref, b_ref, o_ref):
    nf = acc_ref.shape[2] - LANES
    hs = acc_ref[0, :, :nf] + acc_ref[1, :, :nf]
    dn = acc_ref[0, :, nf] + acc_ref[1, :, nf]
    y = hs / dn[:, None] + b_ref[...]
    o_ref[...] = jnp.where(y >= 0.0, y, NEG * y)


def _make_edge_kernel(n_nodes, featp, t_chunks, e_tot):
    mesh = plsc.VectorSubcoreMesh(core_axis_name="c", subcore_axis_name="s")

    @functools.partial(
        pl.kernel,
        mesh=mesh,
        compiler_params=pltpu.CompilerParams(
            needs_layout_passes=False, use_tc_tiling_on_sc=False),
        out_type=jax.ShapeDtypeStruct((NCORE, n_nodes, featp), jnp.float32),
        scratch_types=[
            [pltpu.VMEM((CHUNK,), jnp.int32) for _ in range(NBUF)],
            [pltpu.VMEM((CHUNK,), jnp.int32) for _ in range(NBUF)],
            [pltpu.VMEM((CHUNK,), jnp.int32) for _ in range(NBUF)],
            [pltpu.VMEM((CHUNK, featp), jnp.float32) for _ in range(NBUF)],
            [pltpu.VMEM((CHUNK, LANES), jnp.float32) for _ in range(NBUF)],
            pltpu.VMEM_SHARED((n_nodes, featp), jnp.float32),
            [pltpu.SemaphoreType.DMA for _ in range(NBUF)],
            [pltpu.SemaphoreType.DMA for _ in range(NBUF)],
            [pltpu.SemaphoreType.DMA for _ in range(NBUF)],
        ],
    )
    def edge_kernel(hp_hbm, adw_hbm, idx_hbm, out_hbm,
                    s_v, d_v, ds_v, r_v, b_v, acc, sem_g, sem_i, sem_s):
        c = lax.axis_index("c")
        s = lax.axis_index("s")

        @plsc.parallel_loop(0, CHUNK, 1, unroll=4)
        def _(i):
            for f in range(featp // LANES):
                r_v[0][i, pl.ds(f * LANES, LANES)] = jnp.zeros(
                    (LANES,), jnp.float32)

        npt = n_nodes // NSUB
        nfull = npt // CHUNK
        nrem = npt - nfull * CHUNK
        row0 = s * npt
        for q in range(nfull):
            pltpu.sync_copy(r_v[0], acc.at[pl.ds(row0 + q * CHUNK, CHUNK)])
        if nrem:
            pltpu.sync_copy(r_v[0].at[pl.ds(0, nrem)],
                            acc.at[pl.ds(row0 + nfull * CHUNK, nrem)])
        plsc.subcore_barrier()

        tile_base = (c * NSUB + s) * (t_chunks * CHUNK)

        def issue_gathers(b):
            pltpu.async_copy(hp_hbm.at[s_v[b]], r_v[b], sem_g[b])
            pltpu.async_copy(adw_hbm.at[d_v[b]], b_v[b], sem_g[b])

        def wait_gathers(b):
            pltpu.make_async_copy(hp_hbm.at[s_v[b]], r_v[b], sem_g[b]).wait()
            pltpu.make_async_copy(adw_hbm.at[d_v[b]], b_v[b], sem_g[b]).wait()

        def issue_idx(j, b):
            pltpu.async_copy(idx_hbm.at[0, c, s, j], s_v[b], sem_i[b])
            pltpu.async_copy(idx_hbm.at[1, c, s, j], d_v[b], sem_i[b])

        def wait_idx(j, b):
            pltpu.make_async_copy(idx_hbm.at[0, c, s, j], s_v[b], sem_i[b]).wait()
            pltpu.make_async_copy(idx_hbm.at[1, c, s, j], d_v[b], sem_i[b]).wait()

        def issue_scatter(b):
            pltpu.async_copy(r_v[b], acc.at[ds_v[b]], sem_s[b], add=True)

        def wait_scatter(b):
            pltpu.make_async_copy(r_v[b], acc.at[ds_v[b]], sem_s[b]).wait()

        def compute(j, b):
            gbase = tile_base + j * CHUNK

            @plsc.parallel_loop(0, CHUNK, 1, unroll=4)
            def _(i):
                e = r_v[b][i, pl.ds(featp - LANES, LANES)] + b_v[b][i]
                e = jnp.where(e >= 0.0, e, NEG * e)
                p = jnp.exp(e)
                gv = jnp.full((LANES,), gbase + i, jnp.int32)
                p = jnp.where(gv < e_tot, p, 0.0)
                for f in range((featp - LANES) // LANES):
                    r_v[b][i, pl.ds(f * LANES, LANES)] = (
                        r_v[b][i, pl.ds(f * LANES, LANES)] * p)
                r_v[b][i, pl.ds(featp - LANES, LANES)] = jnp.where(
                    lax.iota(jnp.int32, LANES) == 0, p, 0.0)

        pltpu.sync_copy(idx_hbm.at[0, c, s, 0], s_v[0])
        pltpu.sync_copy(idx_hbm.at[1, c, s, 0], d_v[0])
        issue_gathers(0)
        issue_idx(1, 1)

        nt = t_chunks // NBUF

        def trio_body(jj, carry):
            for b in range(NBUF):
                pn = (b + 1) % NBUF
                pp = (b + 2) % NBUF
                j = jj * NBUF + b
                if b < NBUF - 1:
                    wait_idx(j + 1, pn)
                else:
                    @pl.when(jj < nt - 1)
                    def _():
                        wait_idx(j + 1, pn)
                if b == 0:
                    @pl.when(jj > 0)
                    def _():
                        wait_scatter(pn)
                elif b == 1:
                    @pl.when(jj > 0)
                    def _():
                        wait_scatter(pn)
                else:
                    wait_scatter(pn)
                if b < NBUF - 1:
                    issue_gathers(pn)
                else:
                    @pl.when(jj < nt - 1)
                    def _():
                        issue_gathers(pn)
                if b == 0:
                    issue_idx(j + 2, pp)
                else:
                    @pl.when(jj < nt - 1)
                    def _():
                        issue_idx(j + 2, pp)
                wait_gathers(b)
                for u in range(CHUNK // LANES):
                    ds_v[b][pl.ds(u * LANES, LANES)] = (
                        d_v[b][pl.ds(u * LANES, LANES)])
                compute(j, b)
                issue_scatter(b)
            return carry

        lax.fori_loop(0, nt, trio_body, 0)
        wait_scatter((t_chunks - 2) % NBUF)
        wait_scatter((t_chunks - 1) % NBUF)
        plsc.subcore_barrier()

        pltpu.sync_copy(acc.at[pl.ds(row0, npt)],
                        out_hbm.at[c, pl.ds(row0, npt)])

    return edge_kernel


def kernel(x, edge_index, W, att_src, att_dst, bias):
    n_nodes, in_f = x.shape
    out_f = W.shape[1]
    featp = out_f + LANES
    n_edges = edge_index.shape[1]
    e_tot = n_edges + n_nodes
    n_workers = NCORE * NSUB
    t_chunks = -(-e_tot // (n_workers * CHUNK))
    t_chunks += (-t_chunks) % NBUF
    e_pad = n_workers * t_chunks * CHUNK

    loop = jnp.arange(n_nodes, dtype=jnp.int32)
    zpad = jnp.zeros((e_pad - e_tot,), jnp.int32)
    src = jnp.concatenate([edge_index[0].astype(jnp.int32), loop, zpad])
    dst = jnp.concatenate([edge_index[1].astype(jnp.int32), loop, zpad])
    idx = jnp.stack([src, dst]).reshape(2, NCORE, NSUB, t_chunks, CHUNK)

    hp, adw = pl.pallas_call(
        _prep_body,
        out_shape=[
            jax.ShapeDtypeStruct((n_nodes, featp), jnp.float32),
            jax.ShapeDtypeStruct((n_nodes, LANES), jnp.float32),
        ],
    )(x, W, att_src.reshape(1, out_f), att_dst.reshape(1, out_f))

    acc = _make_edge_kernel(n_nodes, featp, t_chunks, e_tot)(hp, adw, idx)

    grid = 10
    rblk = n_nodes // grid
    out = pl.pallas_call(
        _finish_body,
        grid=(grid,),
        in_specs=[
            pl.BlockSpec((NCORE, rblk, featp), lambda i: (0, i, 0)),
            pl.BlockSpec((1, out_f), lambda i: (0, 0)),
        ],
        out_specs=pl.BlockSpec((rblk, out_f), lambda i: (i, 0)),
        out_shape=jax.ShapeDtypeStruct((n_nodes, out_f), jnp.float32),
    )(acc, bias.reshape(1, out_f))
    return out


# speedup vs baseline: 1.4673x; 1.0348x over previous
import functools

import jax
import jax.numpy as jnp
from jax import lax
from jax.experimental import pallas as pl
from jax.experimental.pallas import tpu as pltpu
from jax.experimental.pallas import tpu_sc as plsc

NEG = 0.2
LANES = 16
CHUNK = 64
NSUB = 16
NCORE = 2
NBUF = 3


def _prep_body(x_ref, w_ref, asr_ref, adr_ref, hp_ref, adw_ref):
    h = jnp.dot(x_ref[...], w_ref[...], preferred_element_type=jnp.float32,
                precision=lax.Precision.HIGHEST)
    asrc = jnp.sum(h * asr_ref[...], axis=1)
    adst = jnp.sum(h * adr_ref[...], axis=1)
    n = h.shape[0]
    hp_ref[...] = jnp.concatenate(
        [h, jnp.broadcast_to(asrc[:, None], (n, LANES))], axis=1)
    adw_ref[...] = jnp.broadcast_to(adst[:, None], (n, LANES))


def _finish_body(acc_ref, b_ref, o_ref):
    nf = acc_ref.shape[2] - LANES
    hs = acc_ref[0, :, :nf] + acc_ref[1, :, :nf]
    dn = acc_ref[0, :, nf] + acc_ref[1, :, nf]
    y = hs / dn[:, None] + b_ref[...]
    o_ref[...] = jnp.where(y >= 0.0, y, NEG * y)


def _make_edge_kernel(n_nodes, featp, t_chunks, e_tot):
    mesh = plsc.VectorSubcoreMesh(core_axis_name="c", subcore_axis_name="s")

    @functools.partial(
        pl.kernel,
        mesh=mesh,
        compiler_params=pltpu.CompilerParams(
            needs_layout_passes=False, use_tc_tiling_on_sc=False),
        out_type=jax.ShapeDtypeStruct((NCORE, n_nodes, featp), jnp.float32),
        scratch_types=[
            [pltpu.VMEM((CHUNK,), jnp.int32) for _ in range(NBUF)],
            [pltpu.VMEM((CHUNK,), jnp.int32) for _ in range(NBUF)],
            [pltpu.VMEM((CHUNK,), jnp.int32) for _ in range(NBUF)],
            [pltpu.VMEM((CHUNK, featp), jnp.float32) for _ in range(NBUF)],
            [pltpu.VMEM((CHUNK, LANES), jnp.float32) for _ in range(NBUF)],
            pltpu.VMEM_SHARED((n_nodes, featp), jnp.float32),
            [pltpu.SemaphoreType.DMA for _ in range(NBUF)],
            [pltpu.SemaphoreType.DMA for _ in range(NBUF)],
            [pltpu.SemaphoreType.DMA for _ in range(NBUF)],
        ],
    )
    def edge_kernel(hp_hbm, adw_hbm, idx_hbm, out_hbm,
                    s_v, d_v, ds_v, r_v, b_v, acc, sem_g, sem_i, sem_s):
        c = lax.axis_index("c")
        s = lax.axis_index("s")

        @plsc.parallel_loop(0, CHUNK, 1, unroll=4)
        def _(i):
            for f in range(featp // LANES):
                r_v[0][i, pl.ds(f * LANES, LANES)] = jnp.zeros(
                    (LANES,), jnp.float32)

        npt = n_nodes // NSUB
        nfull = npt // CHUNK
        nrem = npt - nfull * CHUNK
        row0 = s * npt
        for q in range(nfull):
            pltpu.sync_copy(r_v[0], acc.at[pl.ds(row0 + q * CHUNK, CHUNK)])
        if nrem:
            pltpu.sync_copy(r_v[0].at[pl.ds(0, nrem)],
                            acc.at[pl.ds(row0 + nfull * CHUNK, nrem)])
        plsc.subcore_barrier()

        blk_stride = NCORE * NSUB * CHUNK
        blk_off = (s * NCORE + c) * CHUNK

        def issue_gathers(b):
            pltpu.async_copy(hp_hbm.at[s_v[b]], r_v[b], sem_g[b])
            pltpu.async_copy(adw_hbm.at[d_v[b]], b_v[b], sem_g[b])

        def wait_gathers(b):
            pltpu.make_async_copy(hp_hbm.at[s_v[b]], r_v[b], sem_g[b]).wait()
            pltpu.make_async_copy(adw_hbm.at[d_v[b]], b_v[b], sem_g[b]).wait()

        def issue_idx(j, b):
            pltpu.async_copy(idx_hbm.at[0, c, s, j], s_v[b], sem_i[b])
            pltpu.async_copy(idx_hbm.at[1, c, s, j], d_v[b], sem_i[b])

        def wait_idx(j, b):
            pltpu.make_async_copy(idx_hbm.at[0, c, s, j], s_v[b], sem_i[b]).wait()
            pltpu.make_async_copy(idx_hbm.at[1, c, s, j], d_v[b], sem_i[b]).wait()

        def issue_scatter(b):
            pltpu.async_copy(r_v[b], acc.at[ds_v[b]], sem_s[b], add=True)

        def wait_scatter(b):
            pltpu.make_async_copy(r_v[b], acc.at[ds_v[b]], sem_s[b]).wait()

        def compute(j, b):
            gbase = j * blk_stride + blk_off

            @plsc.parallel_loop(0, CHUNK, 1, unroll=4)
            def _(i):
                e = r_v[b][i, pl.ds(featp - LANES, LANES)] + b_v[b][i]
                e = jnp.where(e >= 0.0, e, NEG * e)
                p = jnp.exp(e)
                gv = jnp.full((LANES,), gbase + i, jnp.int32)
                p = jnp.where(gv < e_tot, p, 0.0)
                for f in range((featp - LANES) // LANES):
                    r_v[b][i, pl.ds(f * LANES, LANES)] = (
                        r_v[b][i, pl.ds(f * LANES, LANES)] * p)
                r_v[b][i, pl.ds(featp - LANES, LANES)] = jnp.where(
                    lax.iota(jnp.int32, LANES) == 0, p, 0.0)

        pltpu.sync_copy(idx_hbm.at[0, c, s, 0], s_v[0])
        pltpu.sync_copy(idx_hbm.at[1, c, s, 0], d_v[0])
        issue_gathers(0)
        issue_idx(1, 1)

        nt = t_chunks // NBUF

        def trio_body(jj, carry):
            for b in range(NBUF):
                pn = (b + 1) % NBUF
                pp = (b + 2) % NBUF
                j = jj * NBUF + b
                if b < NBUF - 1:
                    wait_idx(j + 1, pn)
                else:
                    @pl.when(jj < nt - 1)
                    def _():
                        wait_idx(j + 1, pn)
                if b == 0:
                    @pl.when(jj > 0)
                    def _():
                        wait_scatter(pn)
                elif b == 1:
                    @pl.when(jj > 0)
                    def _():
                        wait_scatter(pn)
                else:
                    wait_scatter(pn)
                if b < NBUF - 1:
                    issue_gathers(pn)
                else:
                    @pl.when(jj < nt - 1)
                    def _():
                        issue_gathers(pn)
                if b == 0:
                    issue_idx(j + 2, pp)
                else:
                    @pl.when(jj < nt - 1)
                    def _():
                        issue_idx(j + 2, pp)
                wait_gathers(b)
                for u in range(CHUNK // LANES):
                    ds_v[b][pl.ds(u * LANES, LANES)] = (
                        d_v[b][pl.ds(u * LANES, LANES)])
                compute(j, b)
                issue_scatter(b)
            return carry

        lax.fori_loop(0, nt, trio_body, 0)
        wait_scatter((t_chunks - 2) % NBUF)
        wait_scatter((t_chunks - 1) % NBUF)
        plsc.subcore_barrier()

        pltpu.sync_copy(acc.at[pl.ds(row0, npt)],
                        out_hbm.at[c, pl.ds(row0, npt)])

    return edge_kernel


def kernel(x, edge_index, W, att_src, att_dst, bias):
    n_nodes, in_f = x.shape
    out_f = W.shape[1]
    featp = out_f + LANES
    n_edges = edge_index.shape[1]
    e_tot = n_edges + n_nodes
    n_workers = NCORE * NSUB
    t_chunks = -(-e_tot // (n_workers * CHUNK))
    t_chunks += (-t_chunks) % NBUF
    e_pad = n_workers * t_chunks * CHUNK

    loop = jnp.arange(n_nodes, dtype=jnp.int32)
    zpad = jnp.zeros((e_pad - e_tot,), jnp.int32)
    src = jnp.concatenate([edge_index[0].astype(jnp.int32), loop, zpad])
    dst = jnp.concatenate([edge_index[1].astype(jnp.int32), loop, zpad])
    idx = (jnp.stack([src, dst])
           .reshape(2, t_chunks, NSUB, NCORE, CHUNK)
           .transpose(0, 3, 2, 1, 4))

    hp, adw = pl.pallas_call(
        _prep_body,
        out_shape=[
            jax.ShapeDtypeStruct((n_nodes, featp), jnp.float32),
            jax.ShapeDtypeStruct((n_nodes, LANES), jnp.float32),
        ],
    )(x, W, att_src.reshape(1, out_f), att_dst.reshape(1, out_f))

    acc = _make_edge_kernel(n_nodes, featp, t_chunks, e_tot)(hp, adw, idx)

    grid = 10
    rblk = n_nodes // grid
    out = pl.pallas_call(
        _finish_body,
        grid=(grid,),
        in_specs=[
            pl.BlockSpec((NCORE, rblk, featp), lambda i: (0, i, 0)),
            pl.BlockSpec((1, out_f), lambda i: (0, 0)),
        ],
        out_specs=pl.BlockSpec((rblk, out_f), lambda i: (i, 0)),
        out_shape=jax.ShapeDtypeStruct((n_nodes, out_f), jnp.float32),
    )(acc, bias.reshape(1, out_f))
    return out


# speedup vs baseline: 1.4682x; 1.0006x over previous
import functools

import jax
import jax.numpy as jnp
from jax import lax
from jax.experimental import pallas as pl
from jax.experimental.pallas import tpu as pltpu
from jax.experimental.pallas import tpu_sc as plsc

NEG = 0.2
LANES = 16
CHUNK = 64
NSUB = 16
NCORE = 2
NBUF = 3


def _prep_body(x_ref, w_ref, asr_ref, adr_ref, hp_ref, adw_ref):
    h = jnp.dot(x_ref[...], w_ref[...], preferred_element_type=jnp.float32,
                precision=lax.Precision.HIGHEST)
    asrc = jnp.sum(h * asr_ref[...], axis=1)
    adst = jnp.sum(h * adr_ref[...], axis=1)
    n = h.shape[0]
    body = jnp.concatenate(
        [h, jnp.broadcast_to(asrc[:, None], (n, LANES))], axis=1)
    fp = body.shape[1]
    pad = jnp.concatenate(
        [jnp.zeros((8, fp - LANES), jnp.float32),
         jnp.full((8, LANES), -1e30, jnp.float32)], axis=1)
    hp_ref[...] = jnp.concatenate([body, pad], axis=0)
    adw_ref[...] = jnp.broadcast_to(adst[:, None], (n, LANES))


def _finish_body(acc_ref, b_ref, o_ref):
    nf = acc_ref.shape[2] - LANES
    hs = acc_ref[0, :, :nf] + acc_ref[1, :, :nf]
    dn = acc_ref[0, :, nf] + acc_ref[1, :, nf]
    y = hs / dn[:, None] + b_ref[...]
    o_ref[...] = jnp.where(y >= 0.0, y, NEG * y)


def _make_edge_kernel(n_nodes, featp, t_chunks, e_tot):
    mesh = plsc.VectorSubcoreMesh(core_axis_name="c", subcore_axis_name="s")

    @functools.partial(
        pl.kernel,
        mesh=mesh,
        compiler_params=pltpu.CompilerParams(
            needs_layout_passes=False, use_tc_tiling_on_sc=False),
        out_type=jax.ShapeDtypeStruct((NCORE, n_nodes, featp), jnp.float32),
        scratch_types=[
            [pltpu.VMEM((2, CHUNK), jnp.int32) for _ in range(NBUF)],
            [pltpu.VMEM((CHUNK,), jnp.int32) for _ in range(NBUF)],
            [pltpu.VMEM((CHUNK, featp), jnp.float32) for _ in range(NBUF)],
            [pltpu.VMEM((CHUNK, LANES), jnp.float32) for _ in range(NBUF)],
            pltpu.VMEM_SHARED((n_nodes, featp), jnp.float32),
            [pltpu.SemaphoreType.DMA for _ in range(NBUF)],
            [pltpu.SemaphoreType.DMA for _ in range(NBUF)],
            [pltpu.SemaphoreType.DMA for _ in range(NBUF)],
        ],
    )
    def edge_kernel(hp_hbm, adw_hbm, idx_hbm, out_hbm,
                    sd_v, ds_v, r_v, b_v, acc, sem_g, sem_i, sem_s):
        c = lax.axis_index("c")
        s = lax.axis_index("s")

        @plsc.parallel_loop(0, CHUNK, 1, unroll=4)
        def _(i):
            for f in range(featp // LANES):
                r_v[0][i, pl.ds(f * LANES, LANES)] = jnp.zeros(
                    (LANES,), jnp.float32)

        npt = n_nodes // NSUB
        nfull = npt // CHUNK
        nrem = npt - nfull * CHUNK
        row0 = s * npt
        for q in range(nfull):
            pltpu.sync_copy(r_v[0], acc.at[pl.ds(row0 + q * CHUNK, CHUNK)])
        if nrem:
            pltpu.sync_copy(r_v[0].at[pl.ds(0, nrem)],
                            acc.at[pl.ds(row0 + nfull * CHUNK, nrem)])
        plsc.subcore_barrier()


        def issue_gathers(b):
            pltpu.async_copy(hp_hbm.at[sd_v[b].at[0]], r_v[b], sem_g[b])
            pltpu.async_copy(adw_hbm.at[sd_v[b].at[1]], b_v[b], sem_g[b])

        def wait_gathers(b):
            pltpu.make_async_copy(
                hp_hbm.at[sd_v[b].at[0]], r_v[b], sem_g[b]).wait()
            pltpu.make_async_copy(
                adw_hbm.at[sd_v[b].at[1]], b_v[b], sem_g[b]).wait()

        def issue_idx(j, b):
            pltpu.async_copy(idx_hbm.at[:, c, s, j], sd_v[b], sem_i[b])

        def wait_idx(j, b):
            pltpu.make_async_copy(
                idx_hbm.at[:, c, s, j], sd_v[b], sem_i[b]).wait()

        def issue_scatter(b):
            pltpu.async_copy(r_v[b], acc.at[ds_v[b]], sem_s[b], add=True)

        def wait_scatter(b):
            pltpu.make_async_copy(r_v[b], acc.at[ds_v[b]], sem_s[b]).wait()

        def compute(j, b):
            @plsc.parallel_loop(0, CHUNK, 1, unroll=8)
            def _(i):
                e = r_v[b][i, pl.ds(featp - LANES, LANES)] + b_v[b][i]
                e = jnp.where(e >= 0.0, e, NEG * e)
                p = jnp.exp(e)
                for f in range((featp - LANES) // LANES):
                    r_v[b][i, pl.ds(f * LANES, LANES)] = (
                        r_v[b][i, pl.ds(f * LANES, LANES)] * p)
                r_v[b][i, pl.ds(featp - LANES, LANES)] = jnp.where(
                    lax.iota(jnp.int32, LANES) == 0, p, 0.0)

        pltpu.sync_copy(idx_hbm.at[:, c, s, 0], sd_v[0])
        issue_gathers(0)
        issue_idx(1, 1)

        nt = t_chunks // NBUF

        def trio_body(jj, carry):
            for b in range(NBUF):
                pn = (b + 1) % NBUF
                pp = (b + 2) % NBUF
                j = jj * NBUF + b
                if b < NBUF - 1:
                    wait_idx(j + 1, pn)
                else:
                    @pl.when(jj < nt - 1)
                    def _():
                        wait_idx(j + 1, pn)
                if b == 0:
                    @pl.when(jj > 0)
                    def _():
                        wait_scatter(pn)
                elif b == 1:
                    @pl.when(jj > 0)
                    def _():
                        wait_scatter(pn)
                else:
                    wait_scatter(pn)
                if b < NBUF - 1:
                    issue_gathers(pn)
                else:
                    @pl.when(jj < nt - 1)
                    def _():
                        issue_gathers(pn)
                if b == 0:
                    issue_idx(j + 2, pp)
                else:
                    @pl.when(jj < nt - 1)
                    def _():
                        issue_idx(j + 2, pp)
                wait_gathers(b)
                for u in range(CHUNK // LANES):
                    ds_v[b][pl.ds(u * LANES, LANES)] = (
                        sd_v[b][1, pl.ds(u * LANES, LANES)])
                compute(j, b)
                issue_scatter(b)
            return carry

        lax.fori_loop(0, nt, trio_body, 0)
        wait_scatter((t_chunks - 2) % NBUF)
        wait_scatter((t_chunks - 1) % NBUF)
        plsc.subcore_barrier()

        pltpu.sync_copy(acc.at[pl.ds(row0, npt)],
                        out_hbm.at[c, pl.ds(row0, npt)])

    return edge_kernel


def kernel(x, edge_index, W, att_src, att_dst, bias):
    n_nodes, in_f = x.shape
    out_f = W.shape[1]
    featp = out_f + LANES
    n_edges = edge_index.shape[1]
    e_tot = n_edges + n_nodes
    n_workers = NCORE * NSUB
    t_chunks = -(-e_tot // (n_workers * CHUNK))
    t_chunks += (-t_chunks) % NBUF
    e_pad = n_workers * t_chunks * CHUNK

    loop = jnp.arange(n_nodes, dtype=jnp.int32)
    spad = jnp.full((e_pad - e_tot,), n_nodes, jnp.int32)
    zpad = jnp.zeros((e_pad - e_tot,), jnp.int32)
    src = jnp.concatenate([edge_index[0].astype(jnp.int32), loop, spad])
    dst = jnp.concatenate([edge_index[1].astype(jnp.int32), loop, zpad])
    idx = (jnp.stack([src, dst])
           .reshape(2, t_chunks, NSUB, NCORE, CHUNK)
           .transpose(0, 3, 2, 1, 4))

    hp, adw = pl.pallas_call(
        _prep_body,
        out_shape=[
            jax.ShapeDtypeStruct((n_nodes + 8, featp), jnp.float32),
            jax.ShapeDtypeStruct((n_nodes, LANES), jnp.float32),
        ],
    )(x, W, att_src.reshape(1, out_f), att_dst.reshape(1, out_f))

    acc = _make_edge_kernel(n_nodes, featp, t_chunks, e_tot)(hp, adw, idx)

    grid = 10
    rblk = n_nodes // grid
    out = pl.pallas_call(
        _finish_body,
        grid=(grid,),
        in_specs=[
            pl.BlockSpec((NCORE, rblk, featp), lambda i: (0, i, 0)),
            pl.BlockSpec((1, out_f), lambda i: (0, 0)),
        ],
        out_specs=pl.BlockSpec((rblk, out_f), lambda i: (i, 0)),
        out_shape=jax.ShapeDtypeStruct((n_nodes, out_f), jnp.float32),
    )(acc, bias.reshape(1, out_f))
    return out
